# baseline (speedup 1.0000x reference)
import functools

import jax
import jax.numpy as jnp
from jax import lax
from jax.experimental import pallas as pl
from jax.experimental.pallas import tpu as pltpu
from jax.experimental.pallas import tpu_sc as plsc


def _hist_sc_packed(idx, k):
    bsz, n_desc = idx.shape
    half = bsz // 2
    info = plsc.get_sparse_core_info()
    nc, ns, L = info.num_cores, info.num_subcores, info.num_lanes
    nw = nc * ns
    prows_per_w = half // nw
    NP = 8
    nbatches = prows_per_w // NP

    mesh = plsc.VectorSubcoreMesh(core_axis_name="c", subcore_axis_name="s")

    @functools.partial(
        pl.kernel,
        mesh=mesh,
        out_type=jax.ShapeDtypeStruct((half, k), jnp.int32),
        compiler_params=pltpu.CompilerParams(needs_layout_passes=False),
        scratch_types=[
            pltpu.VMEM((2 * NP * n_desc,), jnp.int32),
            pltpu.VMEM((2 * NP * n_desc,), jnp.int32),
            pltpu.VMEM((NP * k,), jnp.int32),
            pltpu.VMEM((NP * k,), jnp.int32),
            pltpu.SemaphoreType.DMA,
            pltpu.SemaphoreType.DMA,
            pltpu.SemaphoreType.DMA,
            pltpu.SemaphoreType.DMA,
        ],
    )
    def hist_kernel(idx_hbm, out_hbm, idx_v0, idx_v1,
                    hist_v0, hist_v1, si0, si1, so0, so1):
        wid = lax.axis_index("s") * nc + lax.axis_index("c")
        row0 = wid * prows_per_w
        one_lo = jnp.full((L,), 1, jnp.int32)
        one_hi = jnp.full((L,), 65536, jnp.int32)
        zeros = jnp.zeros((L,), jnp.int32)
        idx_bufs = [idx_v0, idx_v1]
        hist_bufs = [hist_v0, hist_v1]
        in_sems = [si0, si1]
        out_sems = [so0, so1]
        in_handles = [[], []]
        out_handles = [[], []]

        def start_in(t):
            s = t % 2
            ib = idx_bufs[s]
            base = row0 + t * NP
            hnd = []
            for j in range(NP):
                hnd.append(pltpu.async_copy(
                    idx_hbm.at[base + j],
                    ib.at[pl.ds(j * n_desc, n_desc)], in_sems[s]))
                hnd.append(pltpu.async_copy(
                    idx_hbm.at[half + base + j],
                    ib.at[pl.ds((NP + j) * n_desc, n_desc)], in_sems[s]))
            in_handles[s] = hnd

        def zero_hist(s):
            hb = hist_bufs[s]

            @plsc.parallel_loop(0, NP * k // L, unroll=8)
            def _(i, _hb=hb):
                _hb[pl.ds(i * L, L)] = zeros

        start_in(0)
        start_in(1)
        zero_hist(0)
        zero_hist(1)

        sh = (n_desc // L).bit_length() - 1

        for t in range(nbatches):
            s = t % 2
            ib, hb = idx_bufs[s], hist_bufs[s]
            for h in in_handles[s]:
                h.wait()
            if t >= 2:
                for h in out_handles[s]:
                    h.wait()
                zero_hist(s)

            @plsc.parallel_loop(0, NP * n_desc // L, unroll=16)
            def _(i, _ib=ib, _hb=hb, _sh=sh):
                v = _ib[pl.ds(i * L, L)]
                base = (i >> _sh) * k
                plsc.addupdate_scatter(_hb.at[pl.ds(base, k)], [v], one_lo)

            @plsc.parallel_loop(0, NP * n_desc // L, unroll=16)
            def _(i, _ib=ib, _hb=hb, _sh=sh):
                v = _ib[pl.ds(NP * n_desc + i * L, L)]
                base = (i >> _sh) * k
                plsc.addupdate_scatter(_hb.at[pl.ds(base, k)], [v], one_hi)

            out_handles[s] = [
                pltpu.async_copy(hb.at[pl.ds(p * k, k)],
                                 out_hbm.at[row0 + t * NP + p], out_sems[s])
                for p in range(NP)
            ]
            if t + 2 < nbatches:
                start_in(t + 2)

        for s in (0, 1):
            for h in out_handles[s]:
                h.wait()

    return hist_kernel(idx)


def _svm_tc_packed(hist_pack, Wb, B2):
    half, k = hist_pack.shape
    ncls = Wb.shape[0]
    blk = 256
    nblk = half // blk

    def body(hp_ref, w_ref, b_ref, o_ref):
        hsel = pl.program_id(1)
        x = hp_ref[...]
        xl = jnp.bitwise_and(x, 0xFFFF)
        xh = lax.shift_right_logical(x, 16)
        h = jnp.where(hsel == 0, xl, xh).astype(jnp.float32)
        ssq = jnp.sum(h * h, axis=1, keepdims=True)
        inv = 1.0 / (jnp.sqrt(ssq) + 1e-6)
        acc = lax.dot_general(h.astype(jnp.bfloat16), w_ref[...],
                              (((1,), (1,)), ((), ())),
                              preferred_element_type=jnp.float32)
        o_ref[...] = acc * inv + b_ref[...]

    return pl.pallas_call(
        body,
        grid=(nblk, 2),
        in_specs=[
            pl.BlockSpec((blk, k), lambda i, h: (i, 0)),
            pl.BlockSpec((ncls, k), lambda i, h: (0, 0)),
            pl.BlockSpec((1, ncls), lambda i, h: (0, 0)),
        ],
        out_specs=pl.BlockSpec((blk, ncls), lambda i, h: (i + h * nblk, 0)),
        out_shape=jax.ShapeDtypeStruct((2 * half, ncls), jnp.float32),
    )(hist_pack, Wb, B2)


def kernel(idx, W, B):
    k = W.shape[1]
    hist_pack = _hist_sc_packed(idx, k)
    return _svm_tc_packed(hist_pack, W.astype(jnp.bfloat16),
                          B.reshape(1, -1))


# speedup vs baseline: 1.0855x; 1.0855x over previous
import functools

import jax
import jax.numpy as jnp
from jax import lax
from jax.experimental import pallas as pl
from jax.experimental.pallas import tpu as pltpu
from jax.experimental.pallas import tpu_sc as plsc


def _hist_sc_packed(idx, k):
    bsz, n_desc = idx.shape
    half = bsz // 2
    info = plsc.get_sparse_core_info()
    nc, ns, L = info.num_cores, info.num_subcores, info.num_lanes
    nw = nc * ns
    prows_per_w = half // nw
    NP = 8
    nbatches = prows_per_w // NP

    mesh = plsc.VectorSubcoreMesh(core_axis_name="c", subcore_axis_name="s")

    @functools.partial(
        pl.kernel,
        mesh=mesh,
        out_type=jax.ShapeDtypeStruct((half, k), jnp.int32),
        compiler_params=pltpu.CompilerParams(needs_layout_passes=False),
        scratch_types=[
            pltpu.VMEM((2 * NP * n_desc,), jnp.int32),
            pltpu.VMEM((2 * NP * n_desc,), jnp.int32),
            pltpu.VMEM((NP * k,), jnp.int32),
            pltpu.VMEM((NP * k,), jnp.int32),
            pltpu.SemaphoreType.DMA,
            pltpu.SemaphoreType.DMA,
            pltpu.SemaphoreType.DMA,
            pltpu.SemaphoreType.DMA,
        ],
    )
    def hist_kernel(idx_hbm, out_hbm, idx_v0, idx_v1,
                    hist_v0, hist_v1, si0, si1, so0, so1):
        wid = lax.axis_index("s") * nc + lax.axis_index("c")
        row0 = wid * prows_per_w
        one_lo = jnp.full((L,), 1, jnp.int32)
        one_hi = jnp.full((L,), 65536, jnp.int32)
        zeros = jnp.zeros((L,), jnp.int32)
        idx_bufs = [idx_v0, idx_v1]
        hist_bufs = [hist_v0, hist_v1]
        in_sems = [si0, si1]
        out_sems = [so0, so1]
        in_handles = [[], []]
        out_handles = [[], []]

        def start_in(t):
            s = t % 2
            ib = idx_bufs[s]
            base = row0 + t * NP
            hnd = []
            for j in range(NP):
                hnd.append(pltpu.async_copy(
                    idx_hbm.at[base + j],
                    ib.at[pl.ds(j * n_desc, n_desc)], in_sems[s]))
                hnd.append(pltpu.async_copy(
                    idx_hbm.at[half + base + j],
                    ib.at[pl.ds((NP + j) * n_desc, n_desc)], in_sems[s]))
            in_handles[s] = hnd

        def zero_hist(s):
            hb = hist_bufs[s]

            @plsc.parallel_loop(0, NP * k // L, unroll=8)
            def _(i, _hb=hb):
                _hb[pl.ds(i * L, L)] = zeros

        start_in(0)
        start_in(1)
        zero_hist(0)
        zero_hist(1)

        sh = (n_desc // L).bit_length() - 1

        for t in range(nbatches):
            s = t % 2
            ib, hb = idx_bufs[s], hist_bufs[s]
            for h in in_handles[s]:
                h.wait()
            if t >= 2:
                for h in out_handles[s]:
                    h.wait()
                zero_hist(s)

            @plsc.parallel_loop(0, NP * n_desc // L, unroll=16)
            def _(i, _ib=ib, _hb=hb, _sh=sh):
                v = _ib[pl.ds(i * L, L)]
                base = (i >> _sh) * k
                plsc.addupdate_scatter(_hb.at[pl.ds(base, k)], [v], one_lo)

            @plsc.parallel_loop(0, NP * n_desc // L, unroll=16)
            def _(i, _ib=ib, _hb=hb, _sh=sh):
                v = _ib[pl.ds(NP * n_desc + i * L, L)]
                base = (i >> _sh) * k
                plsc.addupdate_scatter(_hb.at[pl.ds(base, k)], [v], one_hi)

            out_handles[s] = [
                pltpu.async_copy(hb.at[pl.ds(p * k, k)],
                                 out_hbm.at[row0 + t * NP + p], out_sems[s])
                for p in range(NP)
            ]
            if t + 2 < nbatches:
                start_in(t + 2)

        for s in (0, 1):
            for h in out_handles[s]:
                h.wait()

    return hist_kernel(idx)


def _svm_tc_packed(hist_pack, Wb, B2):
    half, k = hist_pack.shape
    ncls = Wb.shape[0]
    blk = 256

    def body(hp_ref, w_ref, b_ref, olo_ref, ohi_ref):
        x = hp_ref[...]
        w = w_ref[...]
        b = b_ref[...]
        for h, o_ref in ((jnp.bitwise_and(x, 0xFFFF).astype(jnp.float32),
                          olo_ref),
                         (lax.shift_right_logical(x, 16).astype(jnp.float32),
                          ohi_ref)):
            ssq = jnp.sum(h * h, axis=1, keepdims=True)
            inv = 1.0 / (jnp.sqrt(ssq) + 1e-6)
            acc = lax.dot_general(h.astype(jnp.bfloat16), w,
                                  (((1,), (1,)), ((), ())),
                                  preferred_element_type=jnp.float32)
            o_ref[...] = acc * inv + b

    out_t = jax.ShapeDtypeStruct((half, ncls), jnp.float32)
    return pl.pallas_call(
        body,
        grid=(half // blk,),
        in_specs=[
            pl.BlockSpec((blk, k), lambda i: (i, 0)),
            pl.BlockSpec((ncls, k), lambda i: (0, 0)),
            pl.BlockSpec((1, ncls), lambda i: (0, 0)),
        ],
        out_specs=[pl.BlockSpec((blk, ncls), lambda i: (i, 0)),
                   pl.BlockSpec((blk, ncls), lambda i: (i, 0))],
        out_shape=[out_t, out_t],
    )(hist_pack, Wb, B2)


def kernel(idx, W, B):
    k = W.shape[1]
    hist_pack = _hist_sc_packed(idx, k)
    out_lo, out_hi = _svm_tc_packed(hist_pack, W.astype(jnp.bfloat16),
                                    B.reshape(1, -1))
    return jnp.concatenate([out_lo, out_hi], axis=0)
